# Initial kernel scaffold; baseline (speedup 1.0000x reference)
import functools

import jax
import jax.numpy as jnp
from jax import lax
from jax.experimental import pallas as pl
from jax.experimental.pallas import tpu as pltpu
from jax.experimental.pallas import tpu_sc as plsc

N = 10000
E = 320000
LM = 1280
SP = 100
PE = 64
ND = 128
DEPTH = 2

NPAD = 10240
NC = 2
NS = 16
NW = NC * NS
CHUNK = 128
KCH = 79
EPW = KCH * CHUNK
EPAD = NW * EPW
ERW = 24

_mesh = plsc.VectorSubcoreMesh(core_axis_name="c", subcore_axis_name="s")


def _sc_radial_body(xT_hbm, srcf_hbm, dstf_hbm, rad_out, xv, sv, dv, rv):
    c = lax.axis_index("c")
    s = lax.axis_index("s")
    wid = s * NC + c
    base = wid * EPW
    pltpu.sync_copy(xT_hbm, xv)
    pltpu.sync_copy(srcf_hbm.at[pl.ds(base, EPW)], sv)
    pltpu.sync_copy(dstf_hbm.at[pl.ds(base, EPW)], dv)
    row0 = jnp.zeros((16,), jnp.int32)
    row1 = row0 + 1
    row2 = row0 + 2

    def chunk(j, carry):
        for g in range(8):
            o = j * CHUNK + g * 16
            isv = sv[pl.ds(o, 16)]
            idv = dv[pl.ds(o, 16)]
            dx = plsc.load_gather(xv, [row0, isv]) - plsc.load_gather(xv, [row0, idv])
            dy = plsc.load_gather(xv, [row1, isv]) - plsc.load_gather(xv, [row1, idv])
            dz = plsc.load_gather(xv, [row2, isv]) - plsc.load_gather(xv, [row2, idv])
            rv[pl.ds(g * 16, 16)] = dx * dx + dy * dy + dz * dz
        pltpu.sync_copy(rv, rad_out.at[pl.ds(base + j * CHUNK, CHUNK)])
        return carry

    lax.fori_loop(0, KCH, chunk, 0)


_sc_radial = pl.kernel(
    _sc_radial_body,
    out_type=jax.ShapeDtypeStruct((EPAD,), jnp.float32),
    mesh=_mesh,
    scratch_types=[
        pltpu.VMEM((3, NPAD), jnp.float32),
        pltpu.VMEM((EPW,), jnp.int32),
        pltpu.VMEM((EPW,), jnp.int32),
        pltpu.VMEM((CHUNK,), jnp.float32),
    ],
)


def _sc_gather_body(a_hbm, b_hbm, src3, dst3, g_out, sv, dv, bufa, bufb, sema, semb):
    c = lax.axis_index("c")
    s = lax.axis_index("s")
    wid = s * NC + c
    base = wid * EPW
    pltpu.sync_copy(src3.at[wid], sv)
    pltpu.sync_copy(dst3.at[wid], dv)

    def chunk(j, carry):
        cpa = pltpu.async_copy(a_hbm.at[sv.at[j]], bufa, sema)
        cpb = pltpu.async_copy(b_hbm.at[dv.at[j]], bufb, semb)
        cpa.wait()
        cpb.wait()

        def addrow(r, cc):
            for k in range(8):
                sl = pl.ds(k * 16, 16)
                bufa[r, sl] = bufa[r, sl] + bufb[r, sl]
            return cc

        lax.fori_loop(0, CHUNK, addrow, 0)
        pltpu.sync_copy(bufa, g_out.at[pl.ds(base + j * CHUNK, CHUNK)])
        return carry

    lax.fori_loop(0, KCH, chunk, 0)


_sc_gather = pl.kernel(
    _sc_gather_body,
    out_type=jax.ShapeDtypeStruct((EPAD, ND), jnp.float32),
    mesh=_mesh,
    scratch_types=[
        pltpu.VMEM((KCH, CHUNK), jnp.int32),
        pltpu.VMEM((KCH, CHUNK), jnp.int32),
        pltpu.VMEM((CHUNK, ND), jnp.float32),
        pltpu.VMEM((CHUNK, ND), jnp.float32),
        pltpu.SemaphoreType.DMA,
        pltpu.SemaphoreType.DMA,
    ],
)


def _sc_scatter_body(m_hbm, src3, part_out, accum, sv, buf, zbuf):
    c = lax.axis_index("c")
    s = lax.axis_index("s")
    wid = s * NC + c
    base = wid * EPW
    rps = NPAD // NS

    z = jnp.zeros((16,), jnp.float32)
    for r in range(16):
        for k in range(8):
            zbuf[r, pl.ds(k * 16, 16)] = z

    def zloop(t, carry):
        pltpu.sync_copy(zbuf, accum.at[pl.ds(s * rps + t * 16, 16)])
        return carry

    lax.fori_loop(0, rps // 16, zloop, 0)
    pltpu.sync_copy(src3.at[wid], sv)
    plsc.subcore_barrier()

    def chunk(j, carry):
        pltpu.sync_copy(m_hbm.at[pl.ds(base + j * CHUNK, CHUNK)], buf)
        pltpu.sync_copy(buf, accum.at[sv.at[j]], add=True)
        return carry

    lax.fori_loop(0, KCH, chunk, 0)
    plsc.subcore_barrier()
    pltpu.sync_copy(accum.at[pl.ds(s * rps, rps)],
                    part_out.at[c, pl.ds(s * rps, rps)])


_sc_scatter = pl.kernel(
    _sc_scatter_body,
    out_type=jax.ShapeDtypeStruct((NC, NPAD, ND), jnp.float32),
    mesh=_mesh,
    scratch_types=[
        pltpu.VMEM_SHARED((NPAD, ND), jnp.float32),
        pltpu.VMEM((KCH, CHUNK), jnp.int32),
        pltpu.VMEM((CHUNK, ND), jnp.float32),
        pltpu.VMEM((16, ND), jnp.float32),
    ],
)


def _dot(a, b):
    return jnp.dot(a, b, preferred_element_type=jnp.float32)


def _silu(t):
    return t * jax.nn.sigmoid(t)


def _tc_h_body(hf_ref, ws_ref, wa_ref, wb_ref, bb_ref, h_out, a_out, b_out):
    h = _dot(hf_ref[...], ws_ref[...])
    h_out[...] = h
    a_out[...] = _dot(h, wa_ref[...])
    b_out[...] = _dot(h, wb_ref[...]) + bb_ref[...]


def _tc_h(hf_p, Ws, wa, wb, bb):
    BR = 512
    return pl.pallas_call(
        _tc_h_body,
        grid=(NPAD // BR,),
        in_specs=[
            pl.BlockSpec((BR, LM), lambda i: (i, 0)),
            pl.BlockSpec((LM, ND), lambda i: (0, 0)),
            pl.BlockSpec((ND, ND), lambda i: (0, 0)),
            pl.BlockSpec((ND, ND), lambda i: (0, 0)),
            pl.BlockSpec((1, ND), lambda i: (0, 0)),
        ],
        out_specs=[pl.BlockSpec((BR, ND), lambda i: (i, 0))] * 3,
        out_shape=[jax.ShapeDtypeStruct((NPAD, ND), jnp.float32)] * 3,
    )(hf_p, Ws, wa, wb, bb)


def _tc_eattr_body(sp_ref, po_ref, wsp_ref, wpo_ref, out_ref):
    out_ref[...] = _dot(sp_ref[...], wsp_ref[...]) + _dot(po_ref[...], wpo_ref[...])


def _tc_eattr(spatial, pos, Wsp, Wpo):
    BR = 3200
    return pl.pallas_call(
        _tc_eattr_body,
        grid=(E // BR,),
        in_specs=[
            pl.BlockSpec((BR, SP), lambda i: (i, 0)),
            pl.BlockSpec((BR, PE), lambda i: (i, 0)),
            pl.BlockSpec((SP, 16), lambda i: (0, 0)),
            pl.BlockSpec((PE, 16), lambda i: (0, 0)),
        ],
        out_specs=pl.BlockSpec((BR, 16), lambda i: (i, 0)),
        out_shape=jax.ShapeDtypeStruct((E, 16), jnp.float32),
    )(spatial, pos, Wsp, Wpo)


def _tc_edge_body(g_ref, er_ref, w1e_ref, w2_ref, b2_ref, awt_ref, ab_ref, m_ref):
    t1 = g_ref[...] + _dot(er_ref[...], w1e_ref[...])
    m1 = _silu(t1)
    t2 = _dot(m1, w2_ref[...]) + b2_ref[...]
    m2 = _silu(t2)
    sc = jnp.sum(m2 * awt_ref[...], axis=1, keepdims=True)
    att = jax.nn.sigmoid(sc + ab_ref[...])
    m_ref[...] = m2 * att


def _tc_edge(G, er, w1e, w2, b2, awt, abb):
    BR = 512
    return pl.pallas_call(
        _tc_edge_body,
        grid=(EPAD // BR,),
        in_specs=[
            pl.BlockSpec((BR, ND), lambda i: (i, 0)),
            pl.BlockSpec((BR, ERW), lambda i: (i, 0)),
            pl.BlockSpec((ERW, ND), lambda i: (0, 0)),
            pl.BlockSpec((ND, ND), lambda i: (0, 0)),
            pl.BlockSpec((1, ND), lambda i: (0, 0)),
            pl.BlockSpec((1, ND), lambda i: (0, 0)),
            pl.BlockSpec((1, ND), lambda i: (0, 0)),
        ],
        out_specs=pl.BlockSpec((BR, ND), lambda i: (i, 0)),
        out_shape=jax.ShapeDtypeStruct((EPAD, ND), jnp.float32),
    )(G, er, w1e, w2, b2, awt, abb)


def _tc_node_body(h_ref, p_ref, n1a_ref, n1b_ref, nb1_ref, n2_ref, nb2_ref,
                  *rest):
    h = h_ref[...]
    agg = p_ref[0] + p_ref[1]
    t = _dot(h, n1a_ref[...]) + _dot(agg, n1b_ref[...]) + nb1_ref[...]
    o = _dot(_silu(t), n2_ref[...]) + nb2_ref[...]
    hn = h + o
    if len(rest) == 1:
        rest[0][...] = hn
    else:
        wa_ref, wb_ref, bb_ref, h_out, a_out, b_out = rest
        h_out[...] = hn
        a_out[...] = _dot(hn, wa_ref[...])
        b_out[...] = _dot(hn, wb_ref[...]) + bb_ref[...]


def _tc_node(h, part, n1a, n1b, nb1, n2, nb2, nxt=None):
    BR = 512
    in_specs = [
        pl.BlockSpec((BR, ND), lambda i: (i, 0)),
        pl.BlockSpec((NC, BR, ND), lambda i: (0, i, 0)),
        pl.BlockSpec((ND, ND), lambda i: (0, 0)),
        pl.BlockSpec((ND, ND), lambda i: (0, 0)),
        pl.BlockSpec((1, ND), lambda i: (0, 0)),
        pl.BlockSpec((ND, ND), lambda i: (0, 0)),
        pl.BlockSpec((1, ND), lambda i: (0, 0)),
    ]
    args = [h, part, n1a, n1b, nb1, n2, nb2]
    nouts = 1
    if nxt is not None:
        in_specs += [
            pl.BlockSpec((ND, ND), lambda i: (0, 0)),
            pl.BlockSpec((ND, ND), lambda i: (0, 0)),
            pl.BlockSpec((1, ND), lambda i: (0, 0)),
        ]
        args += list(nxt)
        nouts = 3
    return pl.pallas_call(
        _tc_node_body,
        grid=(NPAD // BR,),
        in_specs=in_specs,
        out_specs=[pl.BlockSpec((BR, ND), lambda i: (i, 0))] * nouts,
        out_shape=[jax.ShapeDtypeStruct((NPAD, ND), jnp.float32)] * nouts,
    )(*args)


def kernel(h_feats, x, edge_index, spatial_attr, positional_attr,
           W_single, W_spatial, W_pos,
           ew1, eb1, ew2, eb2, aw, ab, nw1, nb1, nw2, nb2):
    f32 = jnp.float32
    hf_p = jnp.pad(h_feats, ((0, NPAD - N), (0, 0)))
    xT = jnp.pad(x, ((0, NPAD - N), (0, 0))).T.astype(f32)
    src = edge_index[0].astype(jnp.int32)
    dst = edge_index[1].astype(jnp.int32)
    padv = jnp.full((EPAD - E,), N, jnp.int32)
    srcf = jnp.concatenate([src, padv])
    dstf = jnp.concatenate([dst, padv])
    src3 = srcf.reshape(NW, KCH, CHUNK)
    dst3 = dstf.reshape(NW, KCH, CHUNK)

    rad = _sc_radial(xT, srcf, dstf)
    h, A, B = _tc_h(hf_p, W_single,
                    ew1[0, :ND, :], ew1[0, ND:2 * ND, :], eb1[0][None, :])
    eattr = _tc_eattr(spatial_attr, positional_attr, W_spatial, W_pos)
    er = jnp.zeros((EPAD, ERW), f32)
    er = er.at[:E, :16].set(eattr)
    er = er.at[:, 16].set(rad)

    for l in range(DEPTH):
        w1e = jnp.zeros((ERW, ND), f32)
        w1e = w1e.at[:16, :].set(ew1[l, 2 * ND + 1:, :])
        w1e = w1e.at[16, :].set(ew1[l, 2 * ND, :])
        G = _sc_gather(A, B, src3, dst3)
        m = _tc_edge(G, er, w1e, ew2[l], eb2[l][None, :],
                     aw[l].reshape(1, ND),
                     jnp.broadcast_to(ab[l].reshape(1, 1), (1, ND)))
        part = _sc_scatter(m, src3)
        if l + 1 < DEPTH:
            h, A, B = _tc_node(h, part, nw1[l, :ND, :], nw1[l, ND:, :],
                               nb1[l][None, :], nw2[l], nb2[l][None, :],
                               nxt=(ew1[l + 1, :ND, :], ew1[l + 1, ND:2 * ND, :],
                                    eb1[l + 1][None, :]))
        else:
            h = _tc_node(h, part, nw1[l, :ND, :], nw1[l, ND:, :],
                         nb1[l][None, :], nw2[l], nb2[l][None, :])
    return h[:N]


# speedup vs baseline: 1.7518x; 1.7518x over previous
import functools

import jax
import jax.numpy as jnp
from jax import lax
from jax.experimental import pallas as pl
from jax.experimental.pallas import tpu as pltpu
from jax.experimental.pallas import tpu_sc as plsc

N = 10000
E = 320000
LM = 1280
SP = 100
PE = 64
ND = 128
DEPTH = 2

NPAD = 10240
NC = 2
NS = 16
NW = NC * NS
CHUNK = 128
KCH = 79
EPW = KCH * CHUNK
EPAD = NW * EPW
ERW = 24

@functools.cache
def _mesh():
    return plsc.VectorSubcoreMesh(core_axis_name="c", subcore_axis_name="s",
                                  num_cores=NC, num_subcores=NS)


def _sc_radial_body(x0_hbm, x1_hbm, x2_hbm, srcf_hbm, dstf_hbm, rad_out,
                    xv0, xv1, xv2, sv, dv, rv):
    c = lax.axis_index("c")
    s = lax.axis_index("s")
    wid = s * NC + c
    base = wid * EPW
    pltpu.sync_copy(x0_hbm, xv0)
    pltpu.sync_copy(x1_hbm, xv1)
    pltpu.sync_copy(x2_hbm, xv2)
    pltpu.sync_copy(srcf_hbm.at[pl.ds(base, EPW)], sv)
    pltpu.sync_copy(dstf_hbm.at[pl.ds(base, EPW)], dv)

    def chunk(j, carry):
        for g in range(8):
            o = j * CHUNK + g * 16
            isv = sv[pl.ds(o, 16)]
            idv = dv[pl.ds(o, 16)]
            dx = plsc.load_gather(xv0, [isv]) - plsc.load_gather(xv0, [idv])
            dy = plsc.load_gather(xv1, [isv]) - plsc.load_gather(xv1, [idv])
            dz = plsc.load_gather(xv2, [isv]) - plsc.load_gather(xv2, [idv])
            rv[pl.ds(g * 16, 16)] = dx * dx + dy * dy + dz * dz
        pltpu.sync_copy(rv, rad_out.at[pl.ds(base + j * CHUNK, CHUNK)])
        return carry

    lax.fori_loop(0, KCH, chunk, 0)


@functools.cache
def _sc_radial():
  return pl.kernel(
    _sc_radial_body,
    out_type=jax.ShapeDtypeStruct((EPAD,), jnp.float32),
    mesh=_mesh(),
    compiler_params=pltpu.CompilerParams(needs_layout_passes=False),
    scratch_types=[
        pltpu.VMEM((NPAD,), jnp.float32),
        pltpu.VMEM((NPAD,), jnp.float32),
        pltpu.VMEM((NPAD,), jnp.float32),
        pltpu.VMEM((EPW,), jnp.int32),
        pltpu.VMEM((EPW,), jnp.int32),
        pltpu.VMEM((CHUNK,), jnp.float32),
    ],
  )


def _sc_gather_body(a_hbm, b_hbm, src3, dst3, g_out, sv, dv, bufa, bufb, sema, semb):
    c = lax.axis_index("c")
    s = lax.axis_index("s")
    wid = s * NC + c
    base = wid * EPW
    pltpu.sync_copy(src3.at[wid], sv)
    pltpu.sync_copy(dst3.at[wid], dv)

    def chunk(j, carry):
        cpa = pltpu.async_copy(a_hbm.at[sv.at[j]], bufa, sema)
        cpb = pltpu.async_copy(b_hbm.at[dv.at[j]], bufb, semb)
        cpa.wait()
        cpb.wait()

        def addrow(r, cc):
            for k in range(8):
                sl = pl.ds(k * 16, 16)
                bufa[r, sl] = bufa[r, sl] + bufb[r, sl]
            return cc

        lax.fori_loop(0, CHUNK, addrow, 0)
        pltpu.sync_copy(bufa, g_out.at[pl.ds(base + j * CHUNK, CHUNK)])
        return carry

    lax.fori_loop(0, KCH, chunk, 0)


@functools.cache
def _sc_gather():
  return pl.kernel(
    _sc_gather_body,
    out_type=jax.ShapeDtypeStruct((EPAD, ND), jnp.float32),
    mesh=_mesh(),
    compiler_params=pltpu.CompilerParams(needs_layout_passes=False),
    scratch_types=[
        pltpu.VMEM((KCH, CHUNK), jnp.int32),
        pltpu.VMEM((KCH, CHUNK), jnp.int32),
        pltpu.VMEM((CHUNK, ND), jnp.float32),
        pltpu.VMEM((CHUNK, ND), jnp.float32),
        pltpu.SemaphoreType.DMA,
        pltpu.SemaphoreType.DMA,
    ],
  )


def _sc_scatter_body(m_hbm, src3, part_out, accum, sv, buf, zbuf):
    c = lax.axis_index("c")
    s = lax.axis_index("s")
    wid = s * NC + c
    base = wid * EPW
    rps = NPAD // NS

    z = jnp.zeros((16,), jnp.float32)
    for r in range(16):
        for k in range(8):
            zbuf[r, pl.ds(k * 16, 16)] = z

    def zloop(t, carry):
        pltpu.sync_copy(zbuf, accum.at[pl.ds(s * rps + t * 16, 16)])
        return carry

    lax.fori_loop(0, rps // 16, zloop, 0)
    pltpu.sync_copy(src3.at[wid], sv)
    plsc.subcore_barrier()

    def chunk(j, carry):
        pltpu.sync_copy(m_hbm.at[pl.ds(base + j * CHUNK, CHUNK)], buf)
        pltpu.sync_copy(buf, accum.at[sv.at[j]], add=True)
        return carry

    lax.fori_loop(0, KCH, chunk, 0)
    plsc.subcore_barrier()
    pltpu.sync_copy(accum.at[pl.ds(s * rps, rps)],
                    part_out.at[c, pl.ds(s * rps, rps)])


@functools.cache
def _sc_scatter():
  return pl.kernel(
    _sc_scatter_body,
    out_type=jax.ShapeDtypeStruct((NC, NPAD, ND), jnp.float32),
    mesh=_mesh(),
    compiler_params=pltpu.CompilerParams(needs_layout_passes=False),
    scratch_types=[
        pltpu.VMEM_SHARED((NPAD, ND), jnp.float32),
        pltpu.VMEM((KCH, CHUNK), jnp.int32),
        pltpu.VMEM((CHUNK, ND), jnp.float32),
        pltpu.VMEM((16, ND), jnp.float32),
    ],
  )


def _dot(a, b):
    return jnp.dot(a, b, preferred_element_type=jnp.float32,
                   precision=jax.lax.Precision.HIGHEST)


def _silu(t):
    return t * jax.nn.sigmoid(t)


def _tc_h_body(hf_ref, ws_ref, wa_ref, wb_ref, bb_ref, h_out, a_out, b_out):
    h = _dot(hf_ref[...], ws_ref[...])
    h_out[...] = h
    a_out[...] = _dot(h, wa_ref[...])
    b_out[...] = _dot(h, wb_ref[...]) + bb_ref[...]


def _tc_h(hf_p, Ws, wa, wb, bb):
    BR = 512
    return pl.pallas_call(
        _tc_h_body,
        grid=(NPAD // BR,),
        in_specs=[
            pl.BlockSpec((BR, LM), lambda i: (i, 0)),
            pl.BlockSpec((LM, ND), lambda i: (0, 0)),
            pl.BlockSpec((ND, ND), lambda i: (0, 0)),
            pl.BlockSpec((ND, ND), lambda i: (0, 0)),
            pl.BlockSpec((1, ND), lambda i: (0, 0)),
        ],
        out_specs=[pl.BlockSpec((BR, ND), lambda i: (i, 0))] * 3,
        out_shape=[jax.ShapeDtypeStruct((NPAD, ND), jnp.float32)] * 3,
    )(hf_p, Ws, wa, wb, bb)


def _tc_eattr_body(sp_ref, po_ref, wsp_ref, wpo_ref, out_ref):
    out_ref[...] = _dot(sp_ref[...], wsp_ref[...]) + _dot(po_ref[...], wpo_ref[...])


def _tc_eattr(spatial, pos, Wsp, Wpo):
    BR = 3200
    return pl.pallas_call(
        _tc_eattr_body,
        grid=(E // BR,),
        in_specs=[
            pl.BlockSpec((BR, SP), lambda i: (i, 0)),
            pl.BlockSpec((BR, PE), lambda i: (i, 0)),
            pl.BlockSpec((SP, 16), lambda i: (0, 0)),
            pl.BlockSpec((PE, 16), lambda i: (0, 0)),
        ],
        out_specs=pl.BlockSpec((BR, 16), lambda i: (i, 0)),
        out_shape=jax.ShapeDtypeStruct((E, 16), jnp.float32),
    )(spatial, pos, Wsp, Wpo)


def _tc_edge_body(g_ref, er_ref, w1e_ref, w2_ref, b2_ref, awt_ref, ab_ref, m_ref):
    t1 = g_ref[...] + _dot(er_ref[...], w1e_ref[...])
    m1 = _silu(t1)
    t2 = _dot(m1, w2_ref[...]) + b2_ref[...]
    m2 = _silu(t2)
    sc = jnp.sum(m2 * awt_ref[...], axis=1, keepdims=True)
    att = jax.nn.sigmoid(sc + ab_ref[...])
    m_ref[...] = m2 * att


def _tc_edge(G, er, w1e, w2, b2, awt, abb):
    BR = 512
    return pl.pallas_call(
        _tc_edge_body,
        grid=(EPAD // BR,),
        in_specs=[
            pl.BlockSpec((BR, ND), lambda i: (i, 0)),
            pl.BlockSpec((BR, ERW), lambda i: (i, 0)),
            pl.BlockSpec((ERW, ND), lambda i: (0, 0)),
            pl.BlockSpec((ND, ND), lambda i: (0, 0)),
            pl.BlockSpec((1, ND), lambda i: (0, 0)),
            pl.BlockSpec((1, ND), lambda i: (0, 0)),
            pl.BlockSpec((1, ND), lambda i: (0, 0)),
        ],
        out_specs=pl.BlockSpec((BR, ND), lambda i: (i, 0)),
        out_shape=jax.ShapeDtypeStruct((EPAD, ND), jnp.float32),
    )(G, er, w1e, w2, b2, awt, abb)


def _tc_node_body(h_ref, p_ref, n1a_ref, n1b_ref, nb1_ref, n2_ref, nb2_ref,
                  *rest):
    h = h_ref[...]
    agg = p_ref[0] + p_ref[1]
    t = _dot(h, n1a_ref[...]) + _dot(agg, n1b_ref[...]) + nb1_ref[...]
    o = _dot(_silu(t), n2_ref[...]) + nb2_ref[...]
    hn = h + o
    if len(rest) == 1:
        rest[0][...] = hn
    else:
        wa_ref, wb_ref, bb_ref, h_out, a_out, b_out = rest
        h_out[...] = hn
        a_out[...] = _dot(hn, wa_ref[...])
        b_out[...] = _dot(hn, wb_ref[...]) + bb_ref[...]


def _tc_node(h, part, n1a, n1b, nb1, n2, nb2, nxt=None):
    BR = 512
    in_specs = [
        pl.BlockSpec((BR, ND), lambda i: (i, 0)),
        pl.BlockSpec((NC, BR, ND), lambda i: (0, i, 0)),
        pl.BlockSpec((ND, ND), lambda i: (0, 0)),
        pl.BlockSpec((ND, ND), lambda i: (0, 0)),
        pl.BlockSpec((1, ND), lambda i: (0, 0)),
        pl.BlockSpec((ND, ND), lambda i: (0, 0)),
        pl.BlockSpec((1, ND), lambda i: (0, 0)),
    ]
    args = [h, part, n1a, n1b, nb1, n2, nb2]
    nouts = 1
    if nxt is not None:
        in_specs += [
            pl.BlockSpec((ND, ND), lambda i: (0, 0)),
            pl.BlockSpec((ND, ND), lambda i: (0, 0)),
            pl.BlockSpec((1, ND), lambda i: (0, 0)),
        ]
        args += list(nxt)
        nouts = 3
    out = pl.pallas_call(
        _tc_node_body,
        grid=(NPAD // BR,),
        in_specs=in_specs,
        out_specs=[pl.BlockSpec((BR, ND), lambda i: (i, 0))] * nouts,
        out_shape=[jax.ShapeDtypeStruct((NPAD, ND), jnp.float32)] * nouts,
    )(*args)
    return out[0] if nouts == 1 else out


def kernel(h_feats, x, edge_index, spatial_attr, positional_attr,
           W_single, W_spatial, W_pos,
           ew1, eb1, ew2, eb2, aw, ab, nw1, nb1, nw2, nb2):
    f32 = jnp.float32
    hf_p = jnp.pad(h_feats, ((0, NPAD - N), (0, 0)))
    xp = jnp.pad(x, ((0, NPAD - N), (0, 0))).astype(f32)
    src = edge_index[0].astype(jnp.int32)
    dst = edge_index[1].astype(jnp.int32)
    padv = jnp.full((EPAD - E,), N, jnp.int32)
    srcf = jnp.concatenate([src, padv])
    dstf = jnp.concatenate([dst, padv])
    src3 = srcf.reshape(NW, KCH, CHUNK)
    dst3 = dstf.reshape(NW, KCH, CHUNK)

    rad = _sc_radial()(xp[:, 0], xp[:, 1], xp[:, 2], srcf, dstf)
    h, A, B = _tc_h(hf_p, W_single,
                    ew1[0, :ND, :], ew1[0, ND:2 * ND, :], eb1[0][None, :])
    eattr = _tc_eattr(spatial_attr, positional_attr, W_spatial, W_pos)
    er = jnp.zeros((EPAD, ERW), f32)
    er = er.at[:E, :16].set(eattr)
    er = er.at[:, 16].set(rad)

    for l in range(DEPTH):
        w1e = jnp.zeros((ERW, ND), f32)
        w1e = w1e.at[:16, :].set(ew1[l, 2 * ND + 1:, :])
        w1e = w1e.at[16, :].set(ew1[l, 2 * ND, :])
        G = _sc_gather()(A, B, src3, dst3)
        m = _tc_edge(G, er, w1e, ew2[l], eb2[l][None, :],
                     aw[l].reshape(1, ND),
                     jnp.broadcast_to(ab[l].reshape(1, 1), (1, ND)))
        part = _sc_scatter()(m, src3)
        if l + 1 < DEPTH:
            h, A, B = _tc_node(h, part, nw1[l, :ND, :], nw1[l, ND:, :],
                               nb1[l][None, :], nw2[l], nb2[l][None, :],
                               nxt=(ew1[l + 1, :ND, :], ew1[l + 1, ND:2 * ND, :],
                                    eb1[l + 1][None, :]))
        else:
            h = _tc_node(h, part, nw1[l, :ND, :], nw1[l, ND:, :],
                         nb1[l][None, :], nw2[l], nb2[l][None, :])
    return h[:N]


# speedup vs baseline: 1.8902x; 1.0790x over previous
import functools

import jax
import jax.numpy as jnp
from jax import lax
from jax.experimental import pallas as pl
from jax.experimental.pallas import tpu as pltpu
from jax.experimental.pallas import tpu_sc as plsc

N = 10000
E = 320000
LM = 1280
SP = 100
PE = 64
ND = 128
DEPTH = 2

NPAD = 10240
NC = 2
NS = 16
NW = NC * NS
CHUNK = 128
KCH = 79
EPW = KCH * CHUNK
EPAD = NW * EPW
ERW = 24

@functools.cache
def _mesh():
    return plsc.VectorSubcoreMesh(core_axis_name="c", subcore_axis_name="s",
                                  num_cores=NC, num_subcores=NS)


def _sc_radial_body(x0_hbm, x1_hbm, x2_hbm, srcf_hbm, dstf_hbm, rad_out,
                    xv0, xv1, xv2, sv, dv, rv):
    c = lax.axis_index("c")
    s = lax.axis_index("s")
    wid = s * NC + c
    base = wid * EPW
    pltpu.sync_copy(x0_hbm, xv0)
    pltpu.sync_copy(x1_hbm, xv1)
    pltpu.sync_copy(x2_hbm, xv2)
    pltpu.sync_copy(srcf_hbm.at[pl.ds(base, EPW)], sv)
    pltpu.sync_copy(dstf_hbm.at[pl.ds(base, EPW)], dv)

    def chunk(j, carry):
        for g in range(8):
            o = j * CHUNK + g * 16
            isv = sv[pl.ds(o, 16)]
            idv = dv[pl.ds(o, 16)]
            dx = plsc.load_gather(xv0, [isv]) - plsc.load_gather(xv0, [idv])
            dy = plsc.load_gather(xv1, [isv]) - plsc.load_gather(xv1, [idv])
            dz = plsc.load_gather(xv2, [isv]) - plsc.load_gather(xv2, [idv])
            rv[pl.ds(g * 16, 16)] = dx * dx + dy * dy + dz * dz
        pltpu.sync_copy(rv, rad_out.at[pl.ds(base + j * CHUNK, CHUNK)])
        return carry

    lax.fori_loop(0, KCH, chunk, 0)


@functools.cache
def _sc_radial():
  return pl.kernel(
    _sc_radial_body,
    out_type=jax.ShapeDtypeStruct((EPAD,), jnp.float32),
    mesh=_mesh(),
    compiler_params=pltpu.CompilerParams(needs_layout_passes=False),
    scratch_types=[
        pltpu.VMEM((NPAD,), jnp.float32),
        pltpu.VMEM((NPAD,), jnp.float32),
        pltpu.VMEM((NPAD,), jnp.float32),
        pltpu.VMEM((EPW,), jnp.int32),
        pltpu.VMEM((EPW,), jnp.int32),
        pltpu.VMEM((CHUNK,), jnp.float32),
    ],
  )


def _sc_gather_body(a_hbm, b_hbm, src3, dst3, g_out, sv, dv, bufa, bufb, bufo,
                    semg0, semg1, semw0, semw1):
    c = lax.axis_index("c")
    s = lax.axis_index("s")
    wid = s * NC + c
    base = wid * EPW
    pltpu.sync_copy(src3.at[wid], sv)
    pltpu.sync_copy(dst3.at[wid], dv)
    semg = (semg0, semg1)
    semw = (semw0, semw1)

    def issue_g(j, slot):
        pltpu.async_copy(a_hbm.at[sv.at[j]], bufa.at[slot], semg[slot])
        pltpu.async_copy(b_hbm.at[dv.at[j]], bufb.at[slot], semg[slot])

    def wait_g(slot):
        d = pltpu.make_async_copy(a_hbm.at[sv.at[0]], bufa.at[slot], semg[slot])
        d.wait()
        d.wait()

    def wait_w(slot):
        pltpu.make_async_copy(bufo.at[slot], g_out.at[pl.ds(0, CHUNK)],
                              semw[slot]).wait()

    def step(i, slot):
        @pl.when(i + 1 < KCH)
        def _():
            issue_g(i + 1, 1 - slot)
        wait_g(slot)

        @pl.when(i >= 2)
        def _():
            wait_w(slot)

        def addrow(r, cc):
            for k in range(8):
                sl = pl.ds(k * 16, 16)
                bufo[slot, r, sl] = bufa[slot, r, sl] + bufb[slot, r, sl]
            return cc

        lax.fori_loop(0, CHUNK, addrow, 0)
        pltpu.async_copy(bufo.at[slot], g_out.at[pl.ds(base + i * CHUNK, CHUNK)],
                         semw[slot])

    issue_g(0, 0)

    def pair(p, carry):
        step(2 * p, 0)

        @pl.when(2 * p + 1 < KCH)
        def _():
            step(2 * p + 1, 1)
        return carry

    lax.fori_loop(0, (KCH + 1) // 2, pair, 0)
    wait_w(0)
    wait_w(1)


@functools.cache
def _sc_gather():
  return pl.kernel(
    _sc_gather_body,
    out_type=jax.ShapeDtypeStruct((EPAD, ND), jnp.float32),
    mesh=_mesh(),
    compiler_params=pltpu.CompilerParams(needs_layout_passes=False),
    scratch_types=[
        pltpu.VMEM((KCH, CHUNK), jnp.int32),
        pltpu.VMEM((KCH, CHUNK), jnp.int32),
        pltpu.VMEM((2, CHUNK, ND), jnp.float32),
        pltpu.VMEM((2, CHUNK, ND), jnp.float32),
        pltpu.VMEM((2, CHUNK, ND), jnp.float32),
        pltpu.SemaphoreType.DMA,
        pltpu.SemaphoreType.DMA,
        pltpu.SemaphoreType.DMA,
        pltpu.SemaphoreType.DMA,
    ],
  )


def _sc_scatter_body(m_hbm, src3, part_out, accum, sv, buf, zbuf,
                     semr0, semr1, sems0, sems1):
    c = lax.axis_index("c")
    s = lax.axis_index("s")
    wid = s * NC + c
    base = wid * EPW
    rps = NPAD // NS

    z = jnp.zeros((16,), jnp.float32)
    for r in range(16):
        for k in range(8):
            zbuf[r, pl.ds(k * 16, 16)] = z

    def zloop(t, carry):
        pltpu.sync_copy(zbuf, accum.at[pl.ds(s * rps + t * 16, 16)])
        return carry

    lax.fori_loop(0, rps // 16, zloop, 0)
    pltpu.sync_copy(src3.at[wid], sv)
    plsc.subcore_barrier()
    semr = (semr0, semr1)
    sems = (sems0, sems1)

    def issue_r(j, slot):
        pltpu.async_copy(m_hbm.at[pl.ds(base + j * CHUNK, CHUNK)],
                         buf.at[slot], semr[slot])

    def wait_r(slot):
        pltpu.make_async_copy(m_hbm.at[pl.ds(base, CHUNK)], buf.at[slot],
                              semr[slot]).wait()

    def wait_s(slot):
        pltpu.make_async_copy(buf.at[slot], accum.at[sv.at[0]],
                              sems[slot]).wait()

    def step(i, slot):
        @pl.when(i + 1 < KCH)
        def _():
            @pl.when(i >= 1)
            def _():
                wait_s(1 - slot)
            issue_r(i + 1, 1 - slot)

        wait_r(slot)
        pltpu.async_copy(buf.at[slot], accum.at[sv.at[i]], sems[slot], add=True)

    issue_r(0, 0)

    def pair(p, carry):
        step(2 * p, 0)

        @pl.when(2 * p + 1 < KCH)
        def _():
            step(2 * p + 1, 1)
        return carry

    lax.fori_loop(0, (KCH + 1) // 2, pair, 0)
    wait_s(0)
    wait_s(1)
    plsc.subcore_barrier()
    pltpu.sync_copy(accum.at[pl.ds(s * rps, rps)],
                    part_out.at[c, pl.ds(s * rps, rps)])


@functools.cache
def _sc_scatter():
  return pl.kernel(
    _sc_scatter_body,
    out_type=jax.ShapeDtypeStruct((NC, NPAD, ND), jnp.float32),
    mesh=_mesh(),
    compiler_params=pltpu.CompilerParams(needs_layout_passes=False),
    scratch_types=[
        pltpu.VMEM_SHARED((NPAD, ND), jnp.float32),
        pltpu.VMEM((KCH, CHUNK), jnp.int32),
        pltpu.VMEM((2, CHUNK, ND), jnp.float32),
        pltpu.VMEM((16, ND), jnp.float32),
        pltpu.SemaphoreType.DMA,
        pltpu.SemaphoreType.DMA,
        pltpu.SemaphoreType.DMA,
        pltpu.SemaphoreType.DMA,
    ],
  )


def _dot(a, b):
    return jnp.dot(a, b, preferred_element_type=jnp.float32,
                   precision=jax.lax.Precision.HIGHEST)


def _silu(t):
    return t * jax.nn.sigmoid(t)


def _tc_h_body(hf_ref, ws_ref, wa_ref, wb_ref, bb_ref, h_out, a_out, b_out):
    h = _dot(hf_ref[...], ws_ref[...])
    h_out[...] = h
    a_out[...] = _dot(h, wa_ref[...])
    b_out[...] = _dot(h, wb_ref[...]) + bb_ref[...]


def _tc_h(hf_p, Ws, wa, wb, bb):
    BR = 512
    return pl.pallas_call(
        _tc_h_body,
        grid=(NPAD // BR,),
        in_specs=[
            pl.BlockSpec((BR, LM), lambda i: (i, 0)),
            pl.BlockSpec((LM, ND), lambda i: (0, 0)),
            pl.BlockSpec((ND, ND), lambda i: (0, 0)),
            pl.BlockSpec((ND, ND), lambda i: (0, 0)),
            pl.BlockSpec((1, ND), lambda i: (0, 0)),
        ],
        out_specs=[pl.BlockSpec((BR, ND), lambda i: (i, 0))] * 3,
        out_shape=[jax.ShapeDtypeStruct((NPAD, ND), jnp.float32)] * 3,
    )(hf_p, Ws, wa, wb, bb)


def _tc_eattr_body(sp_ref, po_ref, wsp_ref, wpo_ref, out_ref):
    out_ref[...] = _dot(sp_ref[...], wsp_ref[...]) + _dot(po_ref[...], wpo_ref[...])


def _tc_eattr(spatial, pos, Wsp, Wpo):
    BR = 3200
    return pl.pallas_call(
        _tc_eattr_body,
        grid=(E // BR,),
        in_specs=[
            pl.BlockSpec((BR, SP), lambda i: (i, 0)),
            pl.BlockSpec((BR, PE), lambda i: (i, 0)),
            pl.BlockSpec((SP, 16), lambda i: (0, 0)),
            pl.BlockSpec((PE, 16), lambda i: (0, 0)),
        ],
        out_specs=pl.BlockSpec((BR, 16), lambda i: (i, 0)),
        out_shape=jax.ShapeDtypeStruct((E, 16), jnp.float32),
    )(spatial, pos, Wsp, Wpo)


def _tc_edge_body(g_ref, er_ref, w1e_ref, w2_ref, b2_ref, awt_ref, ab_ref, m_ref):
    t1 = g_ref[...] + _dot(er_ref[...], w1e_ref[...])
    m1 = _silu(t1)
    t2 = _dot(m1, w2_ref[...]) + b2_ref[...]
    m2 = _silu(t2)
    sc = jnp.sum(m2 * awt_ref[...], axis=1, keepdims=True)
    att = jax.nn.sigmoid(sc + ab_ref[...])
    m_ref[...] = m2 * att


def _tc_edge(G, er, w1e, w2, b2, awt, abb):
    BR = 512
    return pl.pallas_call(
        _tc_edge_body,
        grid=(EPAD // BR,),
        in_specs=[
            pl.BlockSpec((BR, ND), lambda i: (i, 0)),
            pl.BlockSpec((BR, ERW), lambda i: (i, 0)),
            pl.BlockSpec((ERW, ND), lambda i: (0, 0)),
            pl.BlockSpec((ND, ND), lambda i: (0, 0)),
            pl.BlockSpec((1, ND), lambda i: (0, 0)),
            pl.BlockSpec((1, ND), lambda i: (0, 0)),
            pl.BlockSpec((1, ND), lambda i: (0, 0)),
        ],
        out_specs=pl.BlockSpec((BR, ND), lambda i: (i, 0)),
        out_shape=jax.ShapeDtypeStruct((EPAD, ND), jnp.float32),
    )(G, er, w1e, w2, b2, awt, abb)


def _tc_node_body(h_ref, p_ref, n1a_ref, n1b_ref, nb1_ref, n2_ref, nb2_ref,
                  *rest):
    h = h_ref[...]
    agg = p_ref[0] + p_ref[1]
    t = _dot(h, n1a_ref[...]) + _dot(agg, n1b_ref[...]) + nb1_ref[...]
    o = _dot(_silu(t), n2_ref[...]) + nb2_ref[...]
    hn = h + o
    if len(rest) == 1:
        rest[0][...] = hn
    else:
        wa_ref, wb_ref, bb_ref, h_out, a_out, b_out = rest
        h_out[...] = hn
        a_out[...] = _dot(hn, wa_ref[...])
        b_out[...] = _dot(hn, wb_ref[...]) + bb_ref[...]


def _tc_node(h, part, n1a, n1b, nb1, n2, nb2, nxt=None):
    BR = 512
    in_specs = [
        pl.BlockSpec((BR, ND), lambda i: (i, 0)),
        pl.BlockSpec((NC, BR, ND), lambda i: (0, i, 0)),
        pl.BlockSpec((ND, ND), lambda i: (0, 0)),
        pl.BlockSpec((ND, ND), lambda i: (0, 0)),
        pl.BlockSpec((1, ND), lambda i: (0, 0)),
        pl.BlockSpec((ND, ND), lambda i: (0, 0)),
        pl.BlockSpec((1, ND), lambda i: (0, 0)),
    ]
    args = [h, part, n1a, n1b, nb1, n2, nb2]
    nouts = 1
    if nxt is not None:
        in_specs += [
            pl.BlockSpec((ND, ND), lambda i: (0, 0)),
            pl.BlockSpec((ND, ND), lambda i: (0, 0)),
            pl.BlockSpec((1, ND), lambda i: (0, 0)),
        ]
        args += list(nxt)
        nouts = 3
    out = pl.pallas_call(
        _tc_node_body,
        grid=(NPAD // BR,),
        in_specs=in_specs,
        out_specs=[pl.BlockSpec((BR, ND), lambda i: (i, 0))] * nouts,
        out_shape=[jax.ShapeDtypeStruct((NPAD, ND), jnp.float32)] * nouts,
    )(*args)
    return out[0] if nouts == 1 else out


def kernel(h_feats, x, edge_index, spatial_attr, positional_attr,
           W_single, W_spatial, W_pos,
           ew1, eb1, ew2, eb2, aw, ab, nw1, nb1, nw2, nb2):
    f32 = jnp.float32
    hf_p = jnp.pad(h_feats, ((0, NPAD - N), (0, 0)))
    xp = jnp.pad(x, ((0, NPAD - N), (0, 0))).astype(f32)
    src = edge_index[0].astype(jnp.int32)
    dst = edge_index[1].astype(jnp.int32)
    padv = jnp.full((EPAD - E,), N, jnp.int32)
    srcf = jnp.concatenate([src, padv])
    dstf = jnp.concatenate([dst, padv])
    src3 = srcf.reshape(NW, KCH, CHUNK)
    dst3 = dstf.reshape(NW, KCH, CHUNK)

    rad = _sc_radial()(xp[:, 0], xp[:, 1], xp[:, 2], srcf, dstf)
    h, A, B = _tc_h(hf_p, W_single,
                    ew1[0, :ND, :], ew1[0, ND:2 * ND, :], eb1[0][None, :])
    eattr = _tc_eattr(spatial_attr, positional_attr, W_spatial, W_pos)
    er = jnp.zeros((EPAD, ERW), f32)
    er = er.at[:E, :16].set(eattr)
    er = er.at[:, 16].set(rad)

    for l in range(DEPTH):
        w1e = jnp.zeros((ERW, ND), f32)
        w1e = w1e.at[:16, :].set(ew1[l, 2 * ND + 1:, :])
        w1e = w1e.at[16, :].set(ew1[l, 2 * ND, :])
        G = _sc_gather()(A, B, src3, dst3)
        m = _tc_edge(G, er, w1e, ew2[l], eb2[l][None, :],
                     aw[l].reshape(1, ND),
                     jnp.broadcast_to(ab[l].reshape(1, 1), (1, ND)))
        part = _sc_scatter()(m, src3)
        if l + 1 < DEPTH:
            h, A, B = _tc_node(h, part, nw1[l, :ND, :], nw1[l, ND:, :],
                               nb1[l][None, :], nw2[l], nb2[l][None, :],
                               nxt=(ew1[l + 1, :ND, :], ew1[l + 1, ND:2 * ND, :],
                                    eb1[l + 1][None, :]))
        else:
            h = _tc_node(h, part, nw1[l, :ND, :], nw1[l, ND:, :],
                         nb1[l][None, :], nw2[l], nb2[l][None, :])
    return h[:N]
